# baseline (speedup 1.0000x reference)
import functools

import jax
import jax.numpy as jnp
from jax import lax
from jax.experimental import pallas as pl
from jax.experimental.pallas import tpu as pltpu
from jax.experimental.pallas import tpu_sc as plsc

L, N, NARY, D, LABEL = 8, 32768, 2, 64, 2
BN = 1024
NB = N // BN
RPAD = N + BN
C = 128


def _sc_info():
    info = plsc.get_sparse_core_info()
    return info.num_cores, info.num_subcores


@functools.lru_cache(maxsize=None)
def _make_gather_lvl():
    nc, ns = _sc_info()
    nw = nc * ns
    per_w = (NARY * N) // nw
    n_sub = per_w // C
    w_per_child = N // per_w
    mesh = plsc.VectorSubcoreMesh(core_axis_name="c", subcore_axis_name="s")

    @functools.partial(
        pl.kernel,
        mesh=mesh,
        out_type=jax.ShapeDtypeStruct((2, N, 2 * D), jnp.bfloat16),
        compiler_params=pltpu.CompilerParams(use_tc_tiling_on_sc=False),
        scratch_types=[
            pltpu.VMEM((n_sub, C), jnp.int32),
            pltpu.VMEM((C, 2 * D), jnp.bfloat16),
            pltpu.VMEM((C, 2 * D), jnp.bfloat16),
            pltpu.SemaphoreType.DMA,
            pltpu.SemaphoreType.DMA,
        ],
    )
    def gather(table_hbm, idx_hbm, out_hbm, idx_v, buf0, buf1, sem0, sem1):
        wid = lax.axis_index("s") * nc + lax.axis_index("c")
        pltpu.sync_copy(idx_hbm.at[pl.ds(wid * n_sub, n_sub)], idx_v)
        child = wid // w_per_child
        out_base = (wid % w_per_child) * per_w
        ccol = pl.ds(child * D, D)

        def start(j, buf, sem):
            pltpu.async_copy(table_hbm.at[idx_v.at[j]], buf, sem)

        def wait(buf, sem):
            pltpu.make_async_copy(table_hbm.at[idx_v.at[0]], buf, sem).wait()

        def drain(j, buf):
            r = pl.ds(out_base + j * C, C)
            pltpu.sync_copy(buf.at[:, pl.ds(0, D)], out_hbm.at[0, r, ccol])
            pltpu.sync_copy(buf.at[:, pl.ds(D, D)], out_hbm.at[1, r, ccol])

        start(0, buf0, sem0)

        def body(jj, carry):
            j0 = jj * 2
            start(j0 + 1, buf1, sem1)
            wait(buf0, sem0)
            drain(j0, buf0)

            @pl.when(j0 + 2 < n_sub)
            def _():
                start(j0 + 2, buf0, sem0)

            wait(buf1, sem1)
            drain(j0 + 1, buf1)
            return carry

        lax.fori_loop(0, n_sub // 2, body, 0)

    return gather


@functools.lru_cache(maxsize=None)
def _make_gather_emb():
    nc, ns = _sc_info()
    nw = nc * ns
    per_w = N // nw
    n_sub = per_w // C
    mesh = plsc.VectorSubcoreMesh(core_axis_name="c", subcore_axis_name="s")

    @functools.partial(
        pl.kernel,
        mesh=mesh,
        out_type=jax.ShapeDtypeStruct((N, 2 * D), jnp.float32),
        compiler_params=pltpu.CompilerParams(use_tc_tiling_on_sc=False),
        scratch_types=[
            pltpu.VMEM((LABEL, n_sub, C), jnp.int32),
            pltpu.VMEM((LABEL, C, D), jnp.float32),
            pltpu.VMEM((LABEL, C, D), jnp.float32),
            pltpu.SemaphoreType.DMA,
            pltpu.SemaphoreType.DMA,
        ],
    )
    def gather(table_hbm, idx_hbm, out_hbm, idx_v, buf0, buf1, sem0, sem1):
        wid = lax.axis_index("s") * nc + lax.axis_index("c")
        pltpu.sync_copy(idx_hbm.at[:, pl.ds(wid * n_sub, n_sub)], idx_v)
        out_base = wid * per_w

        def start(j, buf, sem):
            pltpu.async_copy(table_hbm.at[idx_v.at[0, j]], buf.at[0], sem)
            pltpu.async_copy(table_hbm.at[idx_v.at[1, j]], buf.at[1], sem)

        def wait(buf, sem):
            pltpu.make_async_copy(table_hbm.at[idx_v.at[0, 0]], buf.at[0],
                                  sem).wait()
            pltpu.make_async_copy(table_hbm.at[idx_v.at[0, 0]], buf.at[1],
                                  sem).wait()

        def drain(j, buf):
            r = pl.ds(out_base + j * C, C)
            pltpu.sync_copy(buf.at[0], out_hbm.at[r, pl.ds(0, D)])
            pltpu.sync_copy(buf.at[1], out_hbm.at[r, pl.ds(D, D)])

        start(0, buf0, sem0)

        def body(jj, carry):
            j0 = jj * 2
            start(j0 + 1, buf1, sem1)
            wait(buf0, sem0)
            drain(j0, buf0)

            @pl.when(j0 + 2 < n_sub)
            def _():
                start(j0 + 2, buf0, sem0)

            wait(buf1, sem1)
            drain(j0 + 1, buf1)
            return carry

        lax.fori_loop(0, n_sub // 2, body, 0)

    return gather


def _gather_emb(table, idx2):
    return _make_gather_emb()(table, idx2)


def _gather_lvl(table, idx2):
    return _make_gather_lvl()(table, idx2)


def _dot(a, b):
    return jnp.dot(a, b, preferred_element_type=jnp.float32)


def _emb_x(emb_ref, wl0_ref, wl1_ref, blin_ref):
    e = emb_ref[...]
    return (_dot(e[:, :D], wl0_ref[...]) + _dot(e[:, D:], wl1_ref[...])
            + blin_ref[...])


def _lvl0_body(emb_ref, wl0_ref, wl1_ref, blin_ref, wwe_ref, be_ref, out_ref):
    i = pl.program_id(0)

    @pl.when(i >= NB)
    def _():
        out_ref[...] = jnp.zeros_like(out_ref)

    @pl.when(i < NB)
    def _():
        x = _emb_x(emb_ref, wl0_ref, wl1_ref, blin_ref)
        s = _dot(x, wwe_ref[...]) + be_ref[...]
        sio = jax.nn.sigmoid(s[:, 2 * D:4 * D])
        u_t = jnp.tanh(s[:, 4 * D:])
        nc_ = sio[:, :D] * u_t
        nh = sio[:, D:] * jnp.tanh(nc_)
        out_ref[...] = jnp.concatenate([nh, nc_], axis=1).astype(jnp.bfloat16)


def _lstm_core(emb_ref, g_ref, wl0_ref, wl1_ref, blin_ref, wwe_ref, u_ref,
               be_ref):
    x = _emb_x(emb_ref, wl0_ref, wl1_ref, blin_ref)
    hcat = g_ref[0]
    ccat = g_ref[1].astype(jnp.float32)
    s = _dot(x, wwe_ref[...]) + _dot(hcat, u_ref[...]) + be_ref[...]
    sig = jax.nn.sigmoid(s[:, :4 * D])
    u_t = jnp.tanh(s[:, 4 * D:])
    fc = sig[:, :2 * D] * ccat
    branch = fc[:, :D] + fc[:, D:]
    nc_ = sig[:, 2 * D:3 * D] * u_t + branch
    nh = sig[:, 3 * D:] * jnp.tanh(nc_)
    return x, nh, nc_


def _mid_body(emb_ref, g_ref, wl0_ref, wl1_ref, blin_ref, wwe_ref, u_ref,
              be_ref, out_ref):
    i = pl.program_id(0)

    @pl.when(i >= NB)
    def _():
        out_ref[...] = jnp.zeros_like(out_ref)

    @pl.when(i < NB)
    def _():
        _, nh, nc_ = _lstm_core(emb_ref, g_ref, wl0_ref, wl1_ref, blin_ref,
                                wwe_ref, u_ref, be_ref)
        out_ref[...] = jnp.concatenate([nh, nc_], axis=1).astype(jnp.bfloat16)


def _last_body(emb_ref, g_ref, wl0_ref, wl1_ref, blin_ref, wwe_ref, u_ref,
               be_ref, oh_ref, oc_ref):
    x, nh, nc_ = _lstm_core(emb_ref, g_ref, wl0_ref, wl1_ref, blin_ref,
                            wwe_ref, u_ref, be_ref)
    nh = nh + x
    oh_ref[...] = jnp.broadcast_to(nh[None], (2, BN, D))
    oc_ref[...] = jnp.broadcast_to(nc_[None], (2, BN, D))


def _wspec(shape):
    nd = len(shape)
    return pl.BlockSpec(shape, lambda i: (0,) * nd)


_W_SPECS_X = [_wspec((D, D)), _wspec((D, D)), _wspec((1, D)),
              _wspec((D, 5 * D))]
_W_SPECS_U = [_wspec((2 * D, 5 * D))]
_BE_SPEC = [_wspec((1, 5 * D))]


def _emb_spec(l):
    del l
    return pl.BlockSpec((BN, 2 * D), lambda i: (jnp.minimum(i, NB - 1), 0))


_G_SPEC = pl.BlockSpec((NARY, BN, 2 * D),
                       lambda i: (0, jnp.minimum(i, NB - 1), 0))
_HC_SHAPE = jax.ShapeDtypeStruct((RPAD, 2 * D), jnp.bfloat16)
_HC_SPEC = pl.BlockSpec((BN, 2 * D), lambda i: (i, 0))


def _make_lvl0():
    return pl.pallas_call(
        _lvl0_body,
        grid=(NB + 1,),
        in_specs=[_emb_spec(0)] + _W_SPECS_X + _BE_SPEC,
        out_specs=_HC_SPEC,
        out_shape=_HC_SHAPE,
    )


def _make_mid():
    return pl.pallas_call(
        _mid_body,
        grid=(NB + 1,),
        in_specs=[_emb_spec(0), _G_SPEC] + _W_SPECS_X + _W_SPECS_U + _BE_SPEC,
        out_specs=_HC_SPEC,
        out_shape=_HC_SHAPE,
    )


def _make_last():
    ospec = pl.BlockSpec((2, BN, D), lambda i: (0, i, 0))
    oshape = jax.ShapeDtypeStruct((2, N, D), jnp.float32)
    return pl.pallas_call(
        _last_body,
        grid=(NB,),
        in_specs=[pl.BlockSpec((BN, 2 * D), lambda i: (i, 0)),
                  pl.BlockSpec((NARY, BN, 2 * D), lambda i: (0, i, 0))]
        + _W_SPECS_X + _W_SPECS_U + _BE_SPEC,
        out_specs=[ospec, ospec],
        out_shape=[oshape, oshape],
    )


_lvl0 = _make_lvl0()
_mid = _make_mid()
_last = _make_last()


def kernel(tensor_levels, indice_levels, tree_num, E, W_lin, b_lin, W_w, W_b,
           Uf_w, Uf_b, Uiuo_w, Uiuo_b):
    tl = tensor_levels.astype(jnp.int32)
    il = indice_levels.astype(jnp.int32)

    ef = E.astype(jnp.float32)
    exs = [_gather_emb(ef, tl[l].transpose(1, 0).reshape(LABEL, N // C, C))
           for l in range(L)]

    adjs = [jnp.where(il[l] > 0, il[l] - 1, N).transpose(1, 0)
            .reshape((NARY * N) // C, C) for l in range(L)]

    wl0, wl1 = W_lin[:D], W_lin[D:]
    blin = b_lin.reshape(1, D)
    wf, wi, wu, wo = (W_w[:, :D], W_w[:, D:2 * D], W_w[:, 2 * D:3 * D],
                      W_w[:, 3 * D:])
    wwe = jnp.concatenate([wf, wf, wi, wo, wu], axis=1)
    be = (jnp.concatenate([W_b[:D], W_b[:D], W_b[D:2 * D], W_b[3 * D:],
                           W_b[2 * D:3 * D]])
          + jnp.concatenate([Uf_b, Uiuo_b[:D], Uiuo_b[2 * D:],
                             Uiuo_b[D:2 * D]])).reshape(1, 5 * D)
    ucat = jnp.concatenate([Uf_w, Uiuo_w[:, :D], Uiuo_w[:, 2 * D:],
                            Uiuo_w[:, D:2 * D]], axis=1).astype(jnp.bfloat16)

    hc = _lvl0(exs[0], wl0, wl1, blin, wwe, be)
    for l in range(1, L - 1):
        g = _gather_lvl(hc, adjs[l])
        hc = _mid(exs[l], g, wl0, wl1, blin, wwe, ucat, be)
    g = _gather_lvl(hc, adjs[L - 1])
    hx, cx = _last(exs[L - 1], g, wl0, wl1, blin, wwe, ucat, be)
    return hx, cx


# speedup vs baseline: 1.9256x; 1.9256x over previous
import functools

import jax
import jax.numpy as jnp
from jax import lax
from jax.experimental import pallas as pl
from jax.experimental.pallas import tpu as pltpu
from jax.experimental.pallas import tpu_sc as plsc

L, N, NARY, D, LABEL = 8, 32768, 2, 64, 2
BN = 1024
NB = N // BN
RPAD = N + BN
C = 128


def _sc_info():
    info = plsc.get_sparse_core_info()
    return info.num_cores, info.num_subcores


@functools.lru_cache(maxsize=None)
def _make_gather_lvl():
    nc, ns = _sc_info()
    nw = nc * ns
    per_w = (NARY * N) // nw
    n_sub = per_w // C
    w_per_child = N // per_w
    mesh = plsc.VectorSubcoreMesh(core_axis_name="c", subcore_axis_name="s")

    @functools.partial(
        pl.kernel,
        mesh=mesh,
        out_type=jax.ShapeDtypeStruct((2, N, 2 * D), jnp.float32),
        compiler_params=pltpu.CompilerParams(use_tc_tiling_on_sc=False),
        scratch_types=[
            pltpu.VMEM((n_sub, C), jnp.int32),
            pltpu.VMEM((C, 2 * D), jnp.float32),
            pltpu.VMEM((C, 2 * D), jnp.float32),
            pltpu.SemaphoreType.DMA,
            pltpu.SemaphoreType.DMA,
        ],
    )
    def gather(table_hbm, idx_hbm, out_hbm, idx_v, buf0, buf1, sem0, sem1):
        wid = lax.axis_index("s") * nc + lax.axis_index("c")
        child = wid // w_per_child
        node0 = (wid % w_per_child) * per_w
        pltpu.sync_copy(idx_hbm.at[pl.ds(wid * n_sub, n_sub)], idx_v)
        ccol = pl.ds(child * D, D)

        def start(j, buf, sem):
            pltpu.async_copy(table_hbm.at[idx_v.at[j]], buf, sem)

        def wait(buf, sem):
            pltpu.make_async_copy(table_hbm.at[idx_v.at[0]], buf, sem).wait()

        def drain(j, buf):
            r = pl.ds(node0 + j * C, C)
            pltpu.sync_copy(buf.at[:, pl.ds(0, D)], out_hbm.at[0, r, ccol])
            pltpu.sync_copy(buf.at[:, pl.ds(D, D)], out_hbm.at[1, r, ccol])

        start(0, buf0, sem0)

        def body(jj, carry):
            j0 = jj * 2
            start(j0 + 1, buf1, sem1)
            wait(buf0, sem0)
            drain(j0, buf0)

            @pl.when(j0 + 2 < n_sub)
            def _():
                start(j0 + 2, buf0, sem0)

            wait(buf1, sem1)
            drain(j0 + 1, buf1)
            return carry

        lax.fori_loop(0, n_sub // 2, body, 0)

    return gather


@functools.lru_cache(maxsize=None)
def _make_gather_emb():
    nc, ns = _sc_info()
    nw = nc * ns
    per_w = N // nw
    n_sub = per_w // C
    mesh = plsc.VectorSubcoreMesh(core_axis_name="c", subcore_axis_name="s")

    @functools.partial(
        pl.kernel,
        mesh=mesh,
        out_type=jax.ShapeDtypeStruct((N, 2 * D), jnp.float32),
        compiler_params=pltpu.CompilerParams(use_tc_tiling_on_sc=False),
        scratch_types=[
            pltpu.VMEM((LABEL, n_sub, C), jnp.int32),
            pltpu.VMEM((LABEL, C, D), jnp.float32),
            pltpu.VMEM((LABEL, C, D), jnp.float32),
            pltpu.SemaphoreType.DMA,
            pltpu.SemaphoreType.DMA,
        ],
    )
    def gather(table_hbm, idx_hbm, out_hbm, idx_v, buf0, buf1, sem0, sem1):
        wid = lax.axis_index("s") * nc + lax.axis_index("c")
        node0 = wid * per_w
        pltpu.sync_copy(idx_hbm.at[:, pl.ds(wid * n_sub, n_sub)], idx_v)

        def start(j, buf, sem):
            pltpu.async_copy(table_hbm.at[idx_v.at[0, j]], buf.at[0], sem)
            pltpu.async_copy(table_hbm.at[idx_v.at[1, j]], buf.at[1], sem)

        def wait(buf, sem):
            pltpu.make_async_copy(table_hbm.at[idx_v.at[0, 0]], buf.at[0],
                                  sem).wait()
            pltpu.make_async_copy(table_hbm.at[idx_v.at[0, 0]], buf.at[1],
                                  sem).wait()

        def drain(j, buf):
            r = pl.ds(node0 + j * C, C)
            pltpu.sync_copy(buf.at[0], out_hbm.at[r, pl.ds(0, D)])
            pltpu.sync_copy(buf.at[1], out_hbm.at[r, pl.ds(D, D)])

        start(0, buf0, sem0)

        def body(jj, carry):
            j0 = jj * 2
            start(j0 + 1, buf1, sem1)
            wait(buf0, sem0)
            drain(j0, buf0)

            @pl.when(j0 + 2 < n_sub)
            def _():
                start(j0 + 2, buf0, sem0)

            wait(buf1, sem1)
            drain(j0 + 1, buf1)
            return carry

        lax.fori_loop(0, n_sub // 2, body, 0)

    return gather


def _gather_emb(table, idx2):
    return _make_gather_emb()(table, idx2)


def _gather_lvl(table, idx2):
    return _make_gather_lvl()(table, idx2)


def _dot(a, b):
    return jnp.dot(a, b, preferred_element_type=jnp.float32)


def _emb_x(emb_ref, wl0_ref, wl1_ref, blin_ref):
    e = emb_ref[...]
    return (_dot(e[:, :D], wl0_ref[...]) + _dot(e[:, D:], wl1_ref[...])
            + blin_ref[...])


def _lvl0_body(emb_ref, wl0_ref, wl1_ref, blin_ref, wwe_ref, be_ref, out_ref):
    i = pl.program_id(0)

    @pl.when(i >= NB)
    def _():
        out_ref[...] = jnp.zeros_like(out_ref)

    @pl.when(i < NB)
    def _():
        x = _emb_x(emb_ref, wl0_ref, wl1_ref, blin_ref)
        s = _dot(x, wwe_ref[...]) + be_ref[...]
        sio = jax.nn.sigmoid(s[:, 2 * D:4 * D])
        u_t = jnp.tanh(s[:, 4 * D:])
        nc_ = sio[:, :D] * u_t
        nh = sio[:, D:] * jnp.tanh(nc_)
        out_ref[...] = jnp.concatenate([nh, nc_], axis=1)


def _lstm_core(emb_ref, g_ref, wl0_ref, wl1_ref, blin_ref, wwe_ref, u_ref,
               be_ref):
    x = _emb_x(emb_ref, wl0_ref, wl1_ref, blin_ref)
    hcat = g_ref[0]
    ccat = g_ref[1]
    s = _dot(x, wwe_ref[...]) + _dot(hcat, u_ref[...]) + be_ref[...]
    sig = jax.nn.sigmoid(s[:, :4 * D])
    u_t = jnp.tanh(s[:, 4 * D:])
    fc = sig[:, :2 * D] * ccat
    branch = fc[:, :D] + fc[:, D:]
    nc_ = sig[:, 2 * D:3 * D] * u_t + branch
    nh = sig[:, 3 * D:] * jnp.tanh(nc_)
    return x, nh, nc_


def _mid_body(emb_ref, g_ref, wl0_ref, wl1_ref, blin_ref, wwe_ref, u_ref,
              be_ref, out_ref):
    i = pl.program_id(0)

    @pl.when(i >= NB)
    def _():
        out_ref[...] = jnp.zeros_like(out_ref)

    @pl.when(i < NB)
    def _():
        _, nh, nc_ = _lstm_core(emb_ref, g_ref, wl0_ref, wl1_ref, blin_ref,
                                wwe_ref, u_ref, be_ref)
        out_ref[...] = jnp.concatenate([nh, nc_], axis=1)


def _last_body(emb_ref, g_ref, wl0_ref, wl1_ref, blin_ref, wwe_ref, u_ref,
               be_ref, oh_ref, oc_ref):
    x, nh, nc_ = _lstm_core(emb_ref, g_ref, wl0_ref, wl1_ref, blin_ref,
                            wwe_ref, u_ref, be_ref)
    nh = nh + x
    oh_ref[...] = jnp.broadcast_to(nh[None], (2, BN, D))
    oc_ref[...] = jnp.broadcast_to(nc_[None], (2, BN, D))


def _wspec(shape):
    nd = len(shape)
    return pl.BlockSpec(shape, lambda i: (0,) * nd)


_W_SPECS_X = [_wspec((D, D)), _wspec((D, D)), _wspec((1, D)),
              _wspec((D, 5 * D))]
_W_SPECS_U = [_wspec((2 * D, 5 * D))]
_BE_SPEC = [_wspec((1, 5 * D))]


def _emb_spec(l):
    del l
    return pl.BlockSpec((BN, 2 * D), lambda i: (jnp.minimum(i, NB - 1), 0))


_G_SPEC = pl.BlockSpec((NARY, BN, 2 * D),
                       lambda i: (0, jnp.minimum(i, NB - 1), 0))
_HC_SHAPE = jax.ShapeDtypeStruct((RPAD, 2 * D), jnp.float32)
_HC_SPEC = pl.BlockSpec((BN, 2 * D), lambda i: (i, 0))


def _make_lvl0():
    return pl.pallas_call(
        _lvl0_body,
        grid=(NB + 1,),
        in_specs=[_emb_spec(0)] + _W_SPECS_X + _BE_SPEC,
        out_specs=_HC_SPEC,
        out_shape=_HC_SHAPE,
    )


def _make_mid():
    return pl.pallas_call(
        _mid_body,
        grid=(NB + 1,),
        in_specs=[_emb_spec(0), _G_SPEC] + _W_SPECS_X + _W_SPECS_U + _BE_SPEC,
        out_specs=_HC_SPEC,
        out_shape=_HC_SHAPE,
    )


def _make_last():
    ospec = pl.BlockSpec((2, BN, D), lambda i: (0, i, 0))
    oshape = jax.ShapeDtypeStruct((2, N, D), jnp.float32)
    return pl.pallas_call(
        _last_body,
        grid=(NB,),
        in_specs=[pl.BlockSpec((BN, 2 * D), lambda i: (i, 0)),
                  pl.BlockSpec((NARY, BN, 2 * D), lambda i: (0, i, 0))]
        + _W_SPECS_X + _W_SPECS_U + _BE_SPEC,
        out_specs=[ospec, ospec],
        out_shape=[oshape, oshape],
    )


_lvl0 = _make_lvl0()
_mid = _make_mid()
_last = _make_last()


def kernel(tensor_levels, indice_levels, tree_num, E, W_lin, b_lin, W_w, W_b,
           Uf_w, Uf_b, Uiuo_w, Uiuo_b):
    tl = tensor_levels.astype(jnp.int32)
    il = indice_levels.astype(jnp.int32)

    ef = E.astype(jnp.float32)
    exs = [_gather_emb(ef, tl[l].transpose(1, 0).reshape(LABEL, N // C, C))
           for l in range(L)]

    adjs = [jnp.where(il[l] > 0, il[l] - 1, N).transpose(1, 0)
            .reshape((NARY * N) // C, C) for l in range(L)]

    wl0, wl1 = W_lin[:D], W_lin[D:]
    blin = b_lin.reshape(1, D)
    wf, wi, wu, wo = (W_w[:, :D], W_w[:, D:2 * D], W_w[:, 2 * D:3 * D],
                      W_w[:, 3 * D:])
    wwe = jnp.concatenate([wf, wf, wi, wo, wu], axis=1)
    be = (jnp.concatenate([W_b[:D], W_b[:D], W_b[D:2 * D], W_b[3 * D:],
                           W_b[2 * D:3 * D]])
          + jnp.concatenate([Uf_b, Uiuo_b[:D], Uiuo_b[2 * D:],
                             Uiuo_b[D:2 * D]])).reshape(1, 5 * D)
    ucat = jnp.concatenate([Uf_w, Uiuo_w[:, :D], Uiuo_w[:, 2 * D:],
                            Uiuo_w[:, D:2 * D]], axis=1)

    hc = _lvl0(exs[0], wl0, wl1, blin, wwe, be)
    for l in range(1, L - 1):
        g = _gather_lvl(hc, adjs[l])
        hc = _mid(exs[l], g, wl0, wl1, blin, wwe, ucat, be)
    g = _gather_lvl(hc, adjs[L - 1])
    hx, cx = _last(exs[L - 1], g, wl0, wl1, blin, wwe, ucat, be)
    return hx, cx


# speedup vs baseline: 2.1217x; 1.1018x over previous
import functools

import jax
import jax.numpy as jnp
from jax import lax
from jax.experimental import pallas as pl
from jax.experimental.pallas import tpu as pltpu
from jax.experimental.pallas import tpu_sc as plsc

L, N, NARY, D, LABEL = 8, 32768, 2, 64, 2
BN = 2048
NB = N // BN
RPAD = N + BN
C = 128


def _sc_info():
    info = plsc.get_sparse_core_info()
    return info.num_cores, info.num_subcores


@functools.lru_cache(maxsize=None)
def _make_gather_lvl():
    nc, ns = _sc_info()
    nw = nc * ns
    per_w = (NARY * N) // nw
    n_sub = per_w // C
    w_per_child = N // per_w
    mesh = plsc.VectorSubcoreMesh(core_axis_name="c", subcore_axis_name="s")

    @functools.partial(
        pl.kernel,
        mesh=mesh,
        out_type=jax.ShapeDtypeStruct((2, N, 2 * D), jnp.float32),
        compiler_params=pltpu.CompilerParams(use_tc_tiling_on_sc=False),
        scratch_types=[
            pltpu.VMEM((n_sub, C), jnp.int32),
            pltpu.VMEM((C, 2 * D), jnp.float32),
            pltpu.VMEM((C, 2 * D), jnp.float32),
            pltpu.SemaphoreType.DMA,
            pltpu.SemaphoreType.DMA,
        ],
    )
    def gather(table_hbm, idx_hbm, out_hbm, idx_v, buf0, buf1, sem0, sem1):
        wid = lax.axis_index("s") * nc + lax.axis_index("c")
        child = wid // w_per_child
        node0 = (wid % w_per_child) * per_w
        pltpu.sync_copy(idx_hbm.at[pl.ds(wid * n_sub, n_sub)], idx_v)
        ccol = pl.ds(child * D, D)

        def start(j, buf, sem):
            pltpu.async_copy(table_hbm.at[idx_v.at[j]], buf, sem)

        def wait(buf, sem):
            pltpu.make_async_copy(table_hbm.at[idx_v.at[0]], buf, sem).wait()

        def drain(j, buf):
            r = pl.ds(node0 + j * C, C)
            pltpu.sync_copy(buf.at[:, pl.ds(0, D)], out_hbm.at[0, r, ccol])
            pltpu.sync_copy(buf.at[:, pl.ds(D, D)], out_hbm.at[1, r, ccol])

        start(0, buf0, sem0)

        def body(jj, carry):
            j0 = jj * 2
            start(j0 + 1, buf1, sem1)
            wait(buf0, sem0)
            drain(j0, buf0)

            @pl.when(j0 + 2 < n_sub)
            def _():
                start(j0 + 2, buf0, sem0)

            wait(buf1, sem1)
            drain(j0 + 1, buf1)
            return carry

        lax.fori_loop(0, n_sub // 2, body, 0)

    return gather


@functools.lru_cache(maxsize=None)
def _make_gather_emb():
    nc, ns = _sc_info()
    nw = nc * ns
    per_w = N // nw
    n_sub = per_w // C
    mesh = plsc.VectorSubcoreMesh(core_axis_name="c", subcore_axis_name="s")

    @functools.partial(
        pl.kernel,
        mesh=mesh,
        out_type=jax.ShapeDtypeStruct((N, 2 * D), jnp.float32),
        compiler_params=pltpu.CompilerParams(use_tc_tiling_on_sc=False),
        scratch_types=[
            pltpu.VMEM((LABEL, n_sub, C), jnp.int32),
            pltpu.VMEM((LABEL, C, D), jnp.float32),
            pltpu.VMEM((LABEL, C, D), jnp.float32),
            pltpu.SemaphoreType.DMA,
            pltpu.SemaphoreType.DMA,
        ],
    )
    def gather(table_hbm, idx_hbm, out_hbm, idx_v, buf0, buf1, sem0, sem1):
        wid = lax.axis_index("s") * nc + lax.axis_index("c")
        node0 = wid * per_w
        pltpu.sync_copy(idx_hbm.at[:, pl.ds(wid * n_sub, n_sub)], idx_v)

        def start(j, buf, sem):
            pltpu.async_copy(table_hbm.at[idx_v.at[0, j]], buf.at[0], sem)
            pltpu.async_copy(table_hbm.at[idx_v.at[1, j]], buf.at[1], sem)

        def wait(buf, sem):
            pltpu.make_async_copy(table_hbm.at[idx_v.at[0, 0]], buf.at[0],
                                  sem).wait()
            pltpu.make_async_copy(table_hbm.at[idx_v.at[0, 0]], buf.at[1],
                                  sem).wait()

        def drain(j, buf):
            r = pl.ds(node0 + j * C, C)
            pltpu.sync_copy(buf.at[0], out_hbm.at[r, pl.ds(0, D)])
            pltpu.sync_copy(buf.at[1], out_hbm.at[r, pl.ds(D, D)])

        start(0, buf0, sem0)

        def body(jj, carry):
            j0 = jj * 2
            start(j0 + 1, buf1, sem1)
            wait(buf0, sem0)
            drain(j0, buf0)

            @pl.when(j0 + 2 < n_sub)
            def _():
                start(j0 + 2, buf0, sem0)

            wait(buf1, sem1)
            drain(j0 + 1, buf1)
            return carry

        lax.fori_loop(0, n_sub // 2, body, 0)

    return gather


def _gather_emb(table, idx2):
    return _make_gather_emb()(table, idx2)


def _gather_lvl(table, idx2):
    return _make_gather_lvl()(table, idx2)


def _dot(a, b):
    return jnp.dot(a, b, preferred_element_type=jnp.float32)


def _emb_x(emb_ref, wl0_ref, wl1_ref, blin_ref):
    e = emb_ref[...]
    return (_dot(e[:, :D], wl0_ref[...]) + _dot(e[:, D:], wl1_ref[...])
            + blin_ref[...])


def _lvl0_body(emb_ref, wl0_ref, wl1_ref, blin_ref, wwe_ref, be_ref, out_ref):
    i = pl.program_id(0)

    @pl.when(i >= NB)
    def _():
        out_ref[...] = jnp.zeros_like(out_ref)

    @pl.when(i < NB)
    def _():
        x = _emb_x(emb_ref, wl0_ref, wl1_ref, blin_ref)
        s = _dot(x, wwe_ref[...]) + be_ref[...]
        sio = jax.nn.sigmoid(s[:, 2 * D:4 * D])
        u_t = jnp.tanh(s[:, 4 * D:])
        nc_ = sio[:, :D] * u_t
        nh = sio[:, D:] * jnp.tanh(nc_)
        out_ref[...] = jnp.concatenate([nh, nc_], axis=1)


def _lstm_core(emb_ref, g_ref, wl0_ref, wl1_ref, blin_ref, wwe_ref, u_ref,
               be_ref):
    x = _emb_x(emb_ref, wl0_ref, wl1_ref, blin_ref)
    hcat = g_ref[0]
    ccat = g_ref[1]
    s = _dot(x, wwe_ref[...]) + _dot(hcat, u_ref[...]) + be_ref[...]
    sig = jax.nn.sigmoid(s[:, :4 * D])
    u_t = jnp.tanh(s[:, 4 * D:])
    fc = sig[:, :2 * D] * ccat
    branch = fc[:, :D] + fc[:, D:]
    nc_ = sig[:, 2 * D:3 * D] * u_t + branch
    nh = sig[:, 3 * D:] * jnp.tanh(nc_)
    return x, nh, nc_


def _mid_body(emb_ref, g_ref, wl0_ref, wl1_ref, blin_ref, wwe_ref, u_ref,
              be_ref, out_ref):
    i = pl.program_id(0)

    @pl.when(i >= NB)
    def _():
        out_ref[...] = jnp.zeros_like(out_ref)

    @pl.when(i < NB)
    def _():
        _, nh, nc_ = _lstm_core(emb_ref, g_ref, wl0_ref, wl1_ref, blin_ref,
                                wwe_ref, u_ref, be_ref)
        out_ref[...] = jnp.concatenate([nh, nc_], axis=1)


def _last_body(emb_ref, g_ref, wl0_ref, wl1_ref, blin_ref, wwe_ref, u_ref,
               be_ref, oh_ref, oc_ref):
    x, nh, nc_ = _lstm_core(emb_ref, g_ref, wl0_ref, wl1_ref, blin_ref,
                            wwe_ref, u_ref, be_ref)
    nh = nh + x
    oh_ref[...] = jnp.broadcast_to(nh[None], (2, BN, D))
    oc_ref[...] = jnp.broadcast_to(nc_[None], (2, BN, D))


def _wspec(shape):
    nd = len(shape)
    return pl.BlockSpec(shape, lambda i: (0,) * nd)


_W_SPECS_X = [_wspec((D, D)), _wspec((D, D)), _wspec((1, D)),
              _wspec((D, 5 * D))]
_W_SPECS_U = [_wspec((2 * D, 5 * D))]
_BE_SPEC = [_wspec((1, 5 * D))]


def _emb_spec(l):
    del l
    return pl.BlockSpec((BN, 2 * D), lambda i: (jnp.minimum(i, NB - 1), 0))


_G_SPEC = pl.BlockSpec((NARY, BN, 2 * D),
                       lambda i: (0, jnp.minimum(i, NB - 1), 0))
_HC_SHAPE = jax.ShapeDtypeStruct((RPAD, 2 * D), jnp.float32)
_HC_SPEC = pl.BlockSpec((BN, 2 * D), lambda i: (i, 0))


def _make_lvl0():
    return pl.pallas_call(
        _lvl0_body,
        grid=(NB + 1,),
        in_specs=[_emb_spec(0)] + _W_SPECS_X + _BE_SPEC,
        out_specs=_HC_SPEC,
        out_shape=_HC_SHAPE,
    )


def _make_mid():
    return pl.pallas_call(
        _mid_body,
        grid=(NB + 1,),
        in_specs=[_emb_spec(0), _G_SPEC] + _W_SPECS_X + _W_SPECS_U + _BE_SPEC,
        out_specs=_HC_SPEC,
        out_shape=_HC_SHAPE,
    )


def _make_last():
    ospec = pl.BlockSpec((2, BN, D), lambda i: (0, i, 0))
    oshape = jax.ShapeDtypeStruct((2, N, D), jnp.float32)
    return pl.pallas_call(
        _last_body,
        grid=(NB,),
        in_specs=[pl.BlockSpec((BN, 2 * D), lambda i: (i, 0)),
                  pl.BlockSpec((NARY, BN, 2 * D), lambda i: (0, i, 0))]
        + _W_SPECS_X + _W_SPECS_U + _BE_SPEC,
        out_specs=[ospec, ospec],
        out_shape=[oshape, oshape],
    )


_lvl0 = _make_lvl0()
_mid = _make_mid()
_last = _make_last()


def kernel(tensor_levels, indice_levels, tree_num, E, W_lin, b_lin, W_w, W_b,
           Uf_w, Uf_b, Uiuo_w, Uiuo_b):
    tl = tensor_levels.astype(jnp.int32)
    il = indice_levels.astype(jnp.int32)

    ef = E.astype(jnp.float32)
    exs = [_gather_emb(ef, tl[l].transpose(1, 0).reshape(LABEL, N // C, C))
           for l in range(L)]

    adjs = [jnp.where(il[l] > 0, il[l] - 1, N).transpose(1, 0)
            .reshape((NARY * N) // C, C) for l in range(L)]

    wl0, wl1 = W_lin[:D], W_lin[D:]
    blin = b_lin.reshape(1, D)
    wf, wi, wu, wo = (W_w[:, :D], W_w[:, D:2 * D], W_w[:, 2 * D:3 * D],
                      W_w[:, 3 * D:])
    wwe = jnp.concatenate([wf, wf, wi, wo, wu], axis=1)
    be = (jnp.concatenate([W_b[:D], W_b[:D], W_b[D:2 * D], W_b[3 * D:],
                           W_b[2 * D:3 * D]])
          + jnp.concatenate([Uf_b, Uiuo_b[:D], Uiuo_b[2 * D:],
                             Uiuo_b[D:2 * D]])).reshape(1, 5 * D)
    ucat = jnp.concatenate([Uf_w, Uiuo_w[:, :D], Uiuo_w[:, 2 * D:],
                            Uiuo_w[:, D:2 * D]], axis=1)

    hc = _lvl0(exs[0], wl0, wl1, blin, wwe, be)
    for l in range(1, L - 1):
        g = _gather_lvl(hc, adjs[l])
        hc = _mid(exs[l], g, wl0, wl1, blin, wwe, ucat, be)
    g = _gather_lvl(hc, adjs[L - 1])
    hx, cx = _last(exs[L - 1], g, wl0, wl1, blin, wwe, ucat, be)
    return hx, cx


# speedup vs baseline: 2.1776x; 1.0263x over previous
import functools

import jax
import jax.numpy as jnp
from jax import lax
from jax.experimental import pallas as pl
from jax.experimental.pallas import tpu as pltpu
from jax.experimental.pallas import tpu_sc as plsc

L, N, NARY, D, LABEL = 8, 32768, 2, 64, 2
BN = 4096
NB = N // BN
RPAD = N + BN
C = 128


def _sc_info():
    info = plsc.get_sparse_core_info()
    return info.num_cores, info.num_subcores


@functools.lru_cache(maxsize=None)
def _make_gather_lvl():
    nc, ns = _sc_info()
    nw = nc * ns
    per_w = (NARY * N) // nw
    n_sub = per_w // C
    w_per_child = N // per_w
    mesh = plsc.VectorSubcoreMesh(core_axis_name="c", subcore_axis_name="s")

    @functools.partial(
        pl.kernel,
        mesh=mesh,
        out_type=jax.ShapeDtypeStruct((2, N, 2 * D), jnp.float32),
        compiler_params=pltpu.CompilerParams(use_tc_tiling_on_sc=False),
        scratch_types=[
            pltpu.VMEM((n_sub, C), jnp.int32),
            pltpu.VMEM((C, 2 * D), jnp.float32),
            pltpu.VMEM((C, 2 * D), jnp.float32),
            pltpu.SemaphoreType.DMA,
            pltpu.SemaphoreType.DMA,
        ],
    )
    def gather(table_hbm, idx_hbm, out_hbm, idx_v, buf0, buf1, sem0, sem1):
        wid = lax.axis_index("s") * nc + lax.axis_index("c")
        child = wid // w_per_child
        node0 = (wid % w_per_child) * per_w
        pltpu.sync_copy(idx_hbm.at[pl.ds(wid * n_sub, n_sub)], idx_v)
        ccol = pl.ds(child * D, D)

        def start(j, buf, sem):
            pltpu.async_copy(table_hbm.at[idx_v.at[j]], buf, sem)

        def wait(buf, sem):
            pltpu.make_async_copy(table_hbm.at[idx_v.at[0]], buf, sem).wait()

        def drain(j, buf):
            r = pl.ds(node0 + j * C, C)
            pltpu.sync_copy(buf.at[:, pl.ds(0, D)], out_hbm.at[0, r, ccol])
            pltpu.sync_copy(buf.at[:, pl.ds(D, D)], out_hbm.at[1, r, ccol])

        start(0, buf0, sem0)

        def body(jj, carry):
            j0 = jj * 2
            start(j0 + 1, buf1, sem1)
            wait(buf0, sem0)
            drain(j0, buf0)

            @pl.when(j0 + 2 < n_sub)
            def _():
                start(j0 + 2, buf0, sem0)

            wait(buf1, sem1)
            drain(j0 + 1, buf1)
            return carry

        lax.fori_loop(0, n_sub // 2, body, 0)

    return gather


@functools.lru_cache(maxsize=None)
def _make_gather_emb():
    nc, ns = _sc_info()
    nw = nc * ns
    per_w = N // nw
    n_sub = per_w // C
    mesh = plsc.VectorSubcoreMesh(core_axis_name="c", subcore_axis_name="s")

    @functools.partial(
        pl.kernel,
        mesh=mesh,
        out_type=jax.ShapeDtypeStruct((N, 2 * D), jnp.float32),
        compiler_params=pltpu.CompilerParams(use_tc_tiling_on_sc=False),
        scratch_types=[
            pltpu.VMEM((LABEL, n_sub, C), jnp.int32),
            pltpu.VMEM((LABEL, C, D), jnp.float32),
            pltpu.VMEM((LABEL, C, D), jnp.float32),
            pltpu.SemaphoreType.DMA,
            pltpu.SemaphoreType.DMA,
        ],
    )
    def gather(table_hbm, idx_hbm, out_hbm, idx_v, buf0, buf1, sem0, sem1):
        wid = lax.axis_index("s") * nc + lax.axis_index("c")
        node0 = wid * per_w
        pltpu.sync_copy(idx_hbm.at[:, pl.ds(wid * n_sub, n_sub)], idx_v)

        def start(j, buf, sem):
            pltpu.async_copy(table_hbm.at[idx_v.at[0, j]], buf.at[0], sem)
            pltpu.async_copy(table_hbm.at[idx_v.at[1, j]], buf.at[1], sem)

        def wait(buf, sem):
            pltpu.make_async_copy(table_hbm.at[idx_v.at[0, 0]], buf.at[0],
                                  sem).wait()
            pltpu.make_async_copy(table_hbm.at[idx_v.at[0, 0]], buf.at[1],
                                  sem).wait()

        def drain(j, buf):
            r = pl.ds(node0 + j * C, C)
            pltpu.sync_copy(buf.at[0], out_hbm.at[r, pl.ds(0, D)])
            pltpu.sync_copy(buf.at[1], out_hbm.at[r, pl.ds(D, D)])

        start(0, buf0, sem0)

        def body(jj, carry):
            j0 = jj * 2
            start(j0 + 1, buf1, sem1)
            wait(buf0, sem0)
            drain(j0, buf0)

            @pl.when(j0 + 2 < n_sub)
            def _():
                start(j0 + 2, buf0, sem0)

            wait(buf1, sem1)
            drain(j0 + 1, buf1)
            return carry

        lax.fori_loop(0, n_sub // 2, body, 0)

    return gather


def _gather_emb(table, idx2):
    return _make_gather_emb()(table, idx2)


def _gather_lvl(table, idx2):
    return _make_gather_lvl()(table, idx2)


def _dot(a, b):
    return jnp.dot(a, b, preferred_element_type=jnp.float32)


def _emb_x(emb_ref, wl0_ref, wl1_ref, blin_ref):
    e = emb_ref[...]
    return (_dot(e[:, :D], wl0_ref[...]) + _dot(e[:, D:], wl1_ref[...])
            + blin_ref[...])


def _lvl0_body(emb_ref, wl0_ref, wl1_ref, blin_ref, wwe_ref, be_ref, out_ref):
    i = pl.program_id(0)

    @pl.when(i >= NB)
    def _():
        out_ref[...] = jnp.zeros_like(out_ref)

    @pl.when(i < NB)
    def _():
        x = _emb_x(emb_ref, wl0_ref, wl1_ref, blin_ref)
        s = _dot(x, wwe_ref[...]) + be_ref[...]
        sio = jax.nn.sigmoid(s[:, 2 * D:4 * D])
        u_t = jnp.tanh(s[:, 4 * D:])
        nc_ = sio[:, :D] * u_t
        nh = sio[:, D:] * jnp.tanh(nc_)
        out_ref[...] = jnp.concatenate([nh, nc_], axis=1)


def _lstm_core(emb_ref, g_ref, wl0_ref, wl1_ref, blin_ref, wwe_ref, u_ref,
               be_ref):
    x = _emb_x(emb_ref, wl0_ref, wl1_ref, blin_ref)
    hcat = g_ref[0]
    ccat = g_ref[1]
    s = _dot(x, wwe_ref[...]) + _dot(hcat, u_ref[...]) + be_ref[...]
    sig = jax.nn.sigmoid(s[:, :4 * D])
    u_t = jnp.tanh(s[:, 4 * D:])
    fc = sig[:, :2 * D] * ccat
    branch = fc[:, :D] + fc[:, D:]
    nc_ = sig[:, 2 * D:3 * D] * u_t + branch
    nh = sig[:, 3 * D:] * jnp.tanh(nc_)
    return x, nh, nc_


def _mid_body(emb_ref, g_ref, wl0_ref, wl1_ref, blin_ref, wwe_ref, u_ref,
              be_ref, out_ref):
    i = pl.program_id(0)

    @pl.when(i >= NB)
    def _():
        out_ref[...] = jnp.zeros_like(out_ref)

    @pl.when(i < NB)
    def _():
        _, nh, nc_ = _lstm_core(emb_ref, g_ref, wl0_ref, wl1_ref, blin_ref,
                                wwe_ref, u_ref, be_ref)
        out_ref[...] = jnp.concatenate([nh, nc_], axis=1)


def _last_body(emb_ref, g_ref, wl0_ref, wl1_ref, blin_ref, wwe_ref, u_ref,
               be_ref, oh_ref, oc_ref):
    x, nh, nc_ = _lstm_core(emb_ref, g_ref, wl0_ref, wl1_ref, blin_ref,
                            wwe_ref, u_ref, be_ref)
    nh = nh + x
    oh_ref[...] = jnp.broadcast_to(nh[None], (2, BN, D))
    oc_ref[...] = jnp.broadcast_to(nc_[None], (2, BN, D))


def _wspec(shape):
    nd = len(shape)
    return pl.BlockSpec(shape, lambda i: (0,) * nd)


_W_SPECS_X = [_wspec((D, D)), _wspec((D, D)), _wspec((1, D)),
              _wspec((D, 5 * D))]
_W_SPECS_U = [_wspec((2 * D, 5 * D))]
_BE_SPEC = [_wspec((1, 5 * D))]


def _emb_spec(l):
    del l
    return pl.BlockSpec((BN, 2 * D), lambda i: (jnp.minimum(i, NB - 1), 0))


_G_SPEC = pl.BlockSpec((NARY, BN, 2 * D),
                       lambda i: (0, jnp.minimum(i, NB - 1), 0))
_HC_SHAPE = jax.ShapeDtypeStruct((RPAD, 2 * D), jnp.float32)
_HC_SPEC = pl.BlockSpec((BN, 2 * D), lambda i: (i, 0))


def _make_lvl0():
    return pl.pallas_call(
        _lvl0_body,
        grid=(NB + 1,),
        in_specs=[_emb_spec(0)] + _W_SPECS_X + _BE_SPEC,
        out_specs=_HC_SPEC,
        out_shape=_HC_SHAPE,
    )


def _make_mid():
    return pl.pallas_call(
        _mid_body,
        grid=(NB + 1,),
        in_specs=[_emb_spec(0), _G_SPEC] + _W_SPECS_X + _W_SPECS_U + _BE_SPEC,
        out_specs=_HC_SPEC,
        out_shape=_HC_SHAPE,
    )


def _make_last():
    ospec = pl.BlockSpec((2, BN, D), lambda i: (0, i, 0))
    oshape = jax.ShapeDtypeStruct((2, N, D), jnp.float32)
    return pl.pallas_call(
        _last_body,
        grid=(NB,),
        in_specs=[pl.BlockSpec((BN, 2 * D), lambda i: (i, 0)),
                  pl.BlockSpec((NARY, BN, 2 * D), lambda i: (0, i, 0))]
        + _W_SPECS_X + _W_SPECS_U + _BE_SPEC,
        out_specs=[ospec, ospec],
        out_shape=[oshape, oshape],
    )


_lvl0 = _make_lvl0()
_mid = _make_mid()
_last = _make_last()


def kernel(tensor_levels, indice_levels, tree_num, E, W_lin, b_lin, W_w, W_b,
           Uf_w, Uf_b, Uiuo_w, Uiuo_b):
    tl = tensor_levels.astype(jnp.int32)
    il = indice_levels.astype(jnp.int32)

    ef = E.astype(jnp.float32)
    exs = [_gather_emb(ef, tl[l].transpose(1, 0).reshape(LABEL, N // C, C))
           for l in range(L)]

    adjs = [jnp.where(il[l] > 0, il[l] - 1, N).transpose(1, 0)
            .reshape((NARY * N) // C, C) for l in range(L)]

    wl0, wl1 = W_lin[:D], W_lin[D:]
    blin = b_lin.reshape(1, D)
    wf, wi, wu, wo = (W_w[:, :D], W_w[:, D:2 * D], W_w[:, 2 * D:3 * D],
                      W_w[:, 3 * D:])
    wwe = jnp.concatenate([wf, wf, wi, wo, wu], axis=1)
    be = (jnp.concatenate([W_b[:D], W_b[:D], W_b[D:2 * D], W_b[3 * D:],
                           W_b[2 * D:3 * D]])
          + jnp.concatenate([Uf_b, Uiuo_b[:D], Uiuo_b[2 * D:],
                             Uiuo_b[D:2 * D]])).reshape(1, 5 * D)
    ucat = jnp.concatenate([Uf_w, Uiuo_w[:, :D], Uiuo_w[:, 2 * D:],
                            Uiuo_w[:, D:2 * D]], axis=1)

    hc = _lvl0(exs[0], wl0, wl1, blin, wwe, be)
    for l in range(1, L - 1):
        g = _gather_lvl(hc, adjs[l])
        hc = _mid(exs[l], g, wl0, wl1, blin, wwe, ucat, be)
    g = _gather_lvl(hc, adjs[L - 1])
    hx, cx = _last(exs[L - 1], g, wl0, wl1, blin, wwe, ucat, be)
    return hx, cx
